# Initial kernel scaffold; baseline (speedup 1.0000x reference)
import jax
import jax.numpy as jnp
from jax.experimental import pallas as pl


def kernel(x1, edge_index1, x2, edge_index2, W0, b0, W1, b1, W2, b2, Wf1, bf1, Wf2, bf2):
    raise NotImplementedError("write your pallas kernel here")


# speedup vs baseline: 13.9580x; 13.9580x over previous
import functools

import jax
import jax.numpy as jnp
from jax import lax
from jax.experimental import pallas as pl
from jax.experimental.pallas import tpu as pltpu
from jax.experimental.pallas import tpu_sc as plsc

N = 10000
E = 320000
D = 128
NC, NS, L = 2, 16, 16
NP = 10240
NPR = NP // 128
EP = 323584
EPS = EP // NS
CH = 128
NCHS = EPS // CH
BLK = 2
RPT = NP // NS
RB = 640

def _deg_body(dst_hbm, z_hbm, out_hbm, dst_v, acc_v):
    b = lax.axis_index("c")
    sid = lax.axis_index("s")
    pltpu.sync_copy(dst_hbm.at[b, sid], dst_v)
    pltpu.sync_copy(z_hbm, acc_v)
    ones = jnp.full((L,), 1.0, jnp.float32)

    def body(i, carry):
        idx = dst_v[pl.ds(i * L, L)]
        plsc.addupdate_scatter(acc_v, [idx], ones)
        return carry

    lax.fori_loop(0, EPS // L, body, 0)
    pltpu.sync_copy(acc_v, out_hbm.at[b, sid])


def _svec_body(src_hbm, dst_hbm, dinv_hbm, z_hbm, out_hbm,
               src_v, dst_v, dinv_v, acc_v):
    b = lax.axis_index("c")
    sid = lax.axis_index("s")
    pltpu.sync_copy(src_hbm.at[b, sid], src_v)
    pltpu.sync_copy(dst_hbm.at[b, sid], dst_v)
    pltpu.sync_copy(dinv_hbm.at[b], dinv_v)
    pltpu.sync_copy(z_hbm, acc_v)

    def body(i, carry):
        sl = pl.ds(i * L, L)
        vals = plsc.load_gather(dinv_v, [dst_v[sl]])
        plsc.addupdate_scatter(acc_v, [src_v[sl]], vals)
        return carry

    lax.fori_loop(0, EPS // L, body, 0)
    pltpu.sync_copy(acc_v, out_hbm.at[b, sid])


def _spmm_body(tab_hbm, src_hbm, dst_hbm, z_hbm, out_hbm,
               src_v, dst_v, rows_v, acc_sh, sem):
    b = lax.axis_index("c")
    sid = lax.axis_index("s")
    pltpu.sync_copy(z_hbm, acc_sh.at[pl.ds(sid * RPT, RPT)])
    plsc.subcore_barrier()

    def body(i, carry):
        pltpu.sync_copy(src_hbm.at[b, sid, pl.ds(i * BLK, BLK)], src_v)
        pltpu.sync_copy(dst_hbm.at[b, sid, pl.ds(i * BLK, BLK)], dst_v)
        for j in range(BLK):
            pltpu.async_copy(tab_hbm.at[src_v.at[j]], rows_v, sem).wait()
            pltpu.sync_copy(rows_v, acc_sh.at[dst_v.at[j]], add=True)
        return carry

    lax.fori_loop(0, NCHS // BLK, body, 0)
    plsc.subcore_barrier()
    pltpu.sync_copy(acc_sh.at[pl.ds(sid * RPT, RPT)],
                    out_hbm.at[b, pl.ds(sid * RPT, RPT)])


@functools.cache
def _sc_kernels():
    mesh = plsc.VectorSubcoreMesh(core_axis_name="c", subcore_axis_name="s",
                                  num_cores=NC, num_subcores=NS)
    deg = pl.kernel(
        _deg_body,
        out_type=jax.ShapeDtypeStruct((2, NS, NP), jnp.float32),
        mesh=mesh,
        scratch_types=[
            pltpu.VMEM((EPS,), jnp.int32),
            pltpu.VMEM((NP,), jnp.float32),
        ],
        compiler_params=pltpu.CompilerParams(needs_layout_passes=False),
    )
    svec = pl.kernel(
        _svec_body,
        out_type=jax.ShapeDtypeStruct((2, NS, NP), jnp.float32),
        mesh=mesh,
        scratch_types=[
            pltpu.VMEM((EPS,), jnp.int32),
            pltpu.VMEM((EPS,), jnp.int32),
            pltpu.VMEM((NP,), jnp.float32),
            pltpu.VMEM((NP,), jnp.float32),
        ],
        compiler_params=pltpu.CompilerParams(needs_layout_passes=False),
    )
    spmm = pl.kernel(
        _spmm_body,
        out_type=jax.ShapeDtypeStruct((2, NP, D), jnp.float32),
        mesh=mesh,
        scratch_types=[
            pltpu.VMEM((BLK, CH), jnp.int32),
            pltpu.VMEM((BLK, CH), jnp.int32),
            pltpu.VMEM((CH, D), jnp.float32),
            pltpu.VMEM_SHARED((NP, D), jnp.float32),
            pltpu.SemaphoreType.DMA,
        ],
        compiler_params=pltpu.CompilerParams(needs_layout_passes=False),
    )
    return deg, svec, spmm


def _mm_body(x_ref, w_ref, o_ref):
    o_ref[...] = jnp.dot(x_ref[0], w_ref[...],
                         preferred_element_type=jnp.float32)[None]


def _matmul(x, w):
    return pl.pallas_call(
        _mm_body,
        grid=(2, NP // RB),
        in_specs=[
            pl.BlockSpec((1, RB, D), lambda b, r: (b, r, 0)),
            pl.BlockSpec((D, D), lambda b, r: (0, 0)),
        ],
        out_specs=pl.BlockSpec((1, RB, D), lambda b, r: (b, r, 0)),
        out_shape=jax.ShapeDtypeStruct((2, NP, D), jnp.float32),
    )(x, w)


def _psum_body(p_ref, o_ref, *, to_dinv):
    @pl.when(pl.program_id(0) == 0)
    def _():
        o_ref[...] = jnp.zeros_like(o_ref)

    o_ref[...] += jnp.sum(p_ref[...], axis=1)
    if to_dinv:
        @pl.when(pl.program_id(0) == NS // 8 - 1)
        def _():
            o_ref[...] = lax.rsqrt(o_ref[...] + 1.0)


def _psum(p, to_dinv):
    return pl.pallas_call(
        functools.partial(_psum_body, to_dinv=to_dinv),
        grid=(NS // 8,),
        in_specs=[pl.BlockSpec((2, 8, NP), lambda k: (0, k, 0))],
        out_specs=pl.BlockSpec((2, NP), lambda k: (0, 0)),
        out_shape=jax.ShapeDtypeStruct((2, NP), jnp.float32),
    )(p)


def _reduce_body(g_ref, o_ref):
    @pl.when(pl.program_id(0) == 0)
    def _():
        o_ref[...] = jnp.zeros_like(o_ref)

    o_ref[...] += jnp.sum(g_ref[...], axis=1)


def _colsum(g):
    return pl.pallas_call(
        _reduce_body,
        grid=(NP // RB,),
        in_specs=[pl.BlockSpec((2, RB, D), lambda r: (0, r, 0))],
        out_specs=pl.BlockSpec((2, D), lambda r: (0, 0)),
        out_shape=jax.ShapeDtypeStruct((2, D), jnp.float32),
    )(g)


def _epilogue_body(v_ref, w2_ref, b2_ref, wf1_ref, bf1_ref, wf2_ref, bf2_ref,
                   o_ref):
    u = jnp.dot(v_ref[...] * (1.0 / N), w2_ref[...],
                preferred_element_type=jnp.float32) + b2_ref[...]
    h = jnp.dot(u[0:1], wf1_ref[0:D, :], preferred_element_type=jnp.float32)
    h = h + jnp.dot(u[1:2], wf1_ref[D:2 * D, :],
                    preferred_element_type=jnp.float32)
    h = jnp.maximum(h + bf1_ref[...], 0.0)
    s = jnp.dot(h, wf2_ref[...], preferred_element_type=jnp.float32)
    s = s + bf2_ref[...]
    o_ref[...] = 1.0 / (1.0 + jnp.exp(-s))


def _epilogue(v, w2, b2, wf1, bf1, wf2, bf2):
    return pl.pallas_call(
        _epilogue_body,
        out_shape=jax.ShapeDtypeStruct((1, 1), jnp.float32),
    )(v, w2, b2.reshape(1, D), wf1, bf1.reshape(1, D), wf2,
      bf2.reshape(1, 1))


def _pad_edges(ei):
    src = ei[0]
    dst = ei[1]
    pad = jnp.full((EP - E,), N, dtype=jnp.int32)
    return (jnp.concatenate([src, pad]), jnp.concatenate([dst, pad]))


def kernel(x1, edge_index1, x2, edge_index2,
           W0, b0, W1, b1, W2, b2, Wf1, bf1, Wf2, bf2):
    f32 = jnp.float32
    src1, dst1 = _pad_edges(edge_index1)
    src2, dst2 = _pad_edges(edge_index2)
    src = jnp.stack([src1, src2]).reshape(2, NS, EPS)
    dst = jnp.stack([dst1, dst2]).reshape(2, NS, EPS)
    src_off = jnp.stack([src1, src2 + NP]).reshape(2, NS, NCHS, CH)
    dst_ch = dst.reshape(2, NS, NCHS, CH)

    znp = jnp.zeros((NP,), f32)
    z640 = jnp.zeros((RPT, D), f32)
    _deg_kernel, _svec_kernel, _spmm_kernel = _sc_kernels()

    dinv = _psum(_deg_kernel(dst, znp), to_dinv=True)
    svec = _psum(_svec_kernel(src, dst, dinv, znp), to_dinv=False)
    node_mask = (jnp.arange(NP) < N)[None, :]
    c = jnp.where(node_mask, dinv * (svec + dinv), 0.0)

    X = jnp.zeros((2, NP, D), f32).at[:, :N, :].set(jnp.stack([x1, x2]))
    dcol = dinv[:, :, None]

    t0 = dcol * _matmul(X, W0)
    z0 = _spmm_kernel(t0.reshape(2 * NP, D), src_off, dst_ch, z640)
    h1 = jnp.maximum(dcol * (z0 + t0) + b0[None, None, :], 0.0)

    t1 = dcol * _matmul(h1, W1)
    z1 = _spmm_kernel(t1.reshape(2 * NP, D), src_off, dst_ch, z640)
    h2 = jnp.maximum(dcol * (z1 + t1) + b1[None, None, :], 0.0)

    v = _colsum(c[:, :, None] * h2)
    return _epilogue(v, W2, b2, Wf1, bf1, Wf2, bf2)


# speedup vs baseline: 16.5849x; 1.1882x over previous
import functools

import jax
import jax.numpy as jnp
from jax import lax
from jax.experimental import pallas as pl
from jax.experimental.pallas import tpu as pltpu
from jax.experimental.pallas import tpu_sc as plsc

N = 10000
E = 320000
D = 128
NC, NS, L = 2, 16, 16
NP = 10240
NPR = NP // 128
EP = 323584
EPS = EP // NS
CH = 128
NCHS = EPS // CH
BLK = 2
RPT = NP // NS
RB = 640

def _deg_body(dst_hbm, z_hbm, out_hbm, dst_v, acc_v):
    b = lax.axis_index("c")
    sid = lax.axis_index("s")
    pltpu.sync_copy(dst_hbm.at[b, sid], dst_v)
    pltpu.sync_copy(z_hbm, acc_v)
    ones = jnp.full((L,), 1.0, jnp.float32)

    def body(i, carry):
        idx = dst_v[pl.ds(i * L, L)]
        plsc.addupdate_scatter(acc_v, [idx], ones)
        return carry

    lax.fori_loop(0, EPS // L, body, 0)
    pltpu.sync_copy(acc_v, out_hbm.at[b, sid])


def _svec_body(src_hbm, dst_hbm, dinv_hbm, z_hbm, out_hbm,
               src_v, dst_v, dinv_v, acc_v):
    b = lax.axis_index("c")
    sid = lax.axis_index("s")
    pltpu.sync_copy(src_hbm.at[b, sid], src_v)
    pltpu.sync_copy(dst_hbm.at[b, sid], dst_v)
    pltpu.sync_copy(dinv_hbm.at[b], dinv_v)
    pltpu.sync_copy(z_hbm, acc_v)

    def body(i, carry):
        sl = pl.ds(i * L, L)
        vals = plsc.load_gather(dinv_v, [dst_v[sl]])
        plsc.addupdate_scatter(acc_v, [src_v[sl]], vals)
        return carry

    lax.fori_loop(0, EPS // L, body, 0)
    pltpu.sync_copy(acc_v, out_hbm.at[b, sid])


def _spmm_body(tab_hbm, idx_hbm, z_hbm, out_hbm,
               idx_v, bufa, bufb, acc_sh, ga, gb, sa, sb):
    b = lax.axis_index("c")
    sid = lax.axis_index("s")
    pltpu.sync_copy(z_hbm, acc_sh.at[pl.ds(sid * RPT, RPT)])
    plsc.subcore_barrier()

    pltpu.sync_copy(idx_hbm.at[b, sid, 0], idx_v.at[0])

    def body(i, carry):
        p = lax.rem(i, 2)

        @pl.when(i > 0)
        def _():
            pltpu.make_async_copy(bufa, acc_sh.at[idx_v.at[p, 2]], sa).wait()
            pltpu.make_async_copy(bufb, acc_sh.at[idx_v.at[p, 3]], sb).wait()

        cpa = pltpu.async_copy(tab_hbm.at[idx_v.at[p, 0]], bufa, ga)
        cpb = pltpu.async_copy(tab_hbm.at[idx_v.at[p, 1]], bufb, gb)

        @pl.when(i + 1 < NCHS // 2)
        def _():
            pltpu.sync_copy(idx_hbm.at[b, sid, i + 1], idx_v.at[1 - p])

        cpa.wait()
        pltpu.async_copy(bufa, acc_sh.at[idx_v.at[p, 2]], sa, add=True)
        cpb.wait()
        pltpu.async_copy(bufb, acc_sh.at[idx_v.at[p, 3]], sb, add=True)
        return carry

    lax.fori_loop(0, NCHS // 2, body, 0)
    lastp = (NCHS // 2 - 1) % 2
    pltpu.make_async_copy(bufa, acc_sh.at[idx_v.at[lastp, 2]], sa).wait()
    pltpu.make_async_copy(bufb, acc_sh.at[idx_v.at[lastp, 3]], sb).wait()
    plsc.subcore_barrier()
    pltpu.sync_copy(acc_sh.at[pl.ds(sid * RPT, RPT)],
                    out_hbm.at[b, pl.ds(sid * RPT, RPT)])


@functools.cache
def _sc_kernels():
    mesh = plsc.VectorSubcoreMesh(core_axis_name="c", subcore_axis_name="s",
                                  num_cores=NC, num_subcores=NS)
    deg = pl.kernel(
        _deg_body,
        out_type=jax.ShapeDtypeStruct((2, NS, NP), jnp.float32),
        mesh=mesh,
        scratch_types=[
            pltpu.VMEM((EPS,), jnp.int32),
            pltpu.VMEM((NP,), jnp.float32),
        ],
        compiler_params=pltpu.CompilerParams(needs_layout_passes=False),
    )
    svec = pl.kernel(
        _svec_body,
        out_type=jax.ShapeDtypeStruct((2, NS, NP), jnp.float32),
        mesh=mesh,
        scratch_types=[
            pltpu.VMEM((EPS,), jnp.int32),
            pltpu.VMEM((EPS,), jnp.int32),
            pltpu.VMEM((NP,), jnp.float32),
            pltpu.VMEM((NP,), jnp.float32),
        ],
        compiler_params=pltpu.CompilerParams(needs_layout_passes=False),
    )
    spmm = pl.kernel(
        _spmm_body,
        out_type=jax.ShapeDtypeStruct((2, NP, D), jnp.float32),
        mesh=mesh,
        scratch_types=[
            pltpu.VMEM((2, 4, CH), jnp.int32),
            pltpu.VMEM((CH, D), jnp.float32),
            pltpu.VMEM((CH, D), jnp.float32),
            pltpu.VMEM_SHARED((NP, D), jnp.float32),
            pltpu.SemaphoreType.DMA,
            pltpu.SemaphoreType.DMA,
            pltpu.SemaphoreType.DMA,
            pltpu.SemaphoreType.DMA,
        ],
        compiler_params=pltpu.CompilerParams(needs_layout_passes=False),
    )
    return deg, svec, spmm


def _mm_body(x_ref, w_ref, o_ref):
    o_ref[...] = jnp.dot(x_ref[0], w_ref[...],
                         preferred_element_type=jnp.float32)[None]


def _matmul(x, w):
    return pl.pallas_call(
        _mm_body,
        grid=(2, NP // RB),
        in_specs=[
            pl.BlockSpec((1, RB, D), lambda b, r: (b, r, 0)),
            pl.BlockSpec((D, D), lambda b, r: (0, 0)),
        ],
        out_specs=pl.BlockSpec((1, RB, D), lambda b, r: (b, r, 0)),
        out_shape=jax.ShapeDtypeStruct((2, NP, D), jnp.float32),
    )(x, w)


def _psum_body(p_ref, o_ref, *, to_dinv):
    @pl.when(pl.program_id(0) == 0)
    def _():
        o_ref[...] = jnp.zeros_like(o_ref)

    o_ref[...] += jnp.sum(p_ref[...], axis=1)
    if to_dinv:
        @pl.when(pl.program_id(0) == NS // 8 - 1)
        def _():
            o_ref[...] = lax.rsqrt(o_ref[...] + 1.0)


def _psum(p, to_dinv):
    return pl.pallas_call(
        functools.partial(_psum_body, to_dinv=to_dinv),
        grid=(NS // 8,),
        in_specs=[pl.BlockSpec((2, 8, NP), lambda k: (0, k, 0))],
        out_specs=pl.BlockSpec((2, NP), lambda k: (0, 0)),
        out_shape=jax.ShapeDtypeStruct((2, NP), jnp.float32),
    )(p)


def _reduce_body(g_ref, o_ref):
    @pl.when(pl.program_id(0) == 0)
    def _():
        o_ref[...] = jnp.zeros_like(o_ref)

    o_ref[...] += jnp.sum(g_ref[...], axis=1)


def _colsum(g):
    return pl.pallas_call(
        _reduce_body,
        grid=(NP // RB,),
        in_specs=[pl.BlockSpec((2, RB, D), lambda r: (0, r, 0))],
        out_specs=pl.BlockSpec((2, D), lambda r: (0, 0)),
        out_shape=jax.ShapeDtypeStruct((2, D), jnp.float32),
    )(g)


def _epilogue_body(v_ref, w2_ref, b2_ref, wf1_ref, bf1_ref, wf2_ref, bf2_ref,
                   o_ref):
    u = jnp.dot(v_ref[...] * (1.0 / N), w2_ref[...],
                preferred_element_type=jnp.float32) + b2_ref[...]
    h = jnp.dot(u[0:1], wf1_ref[0:D, :], preferred_element_type=jnp.float32)
    h = h + jnp.dot(u[1:2], wf1_ref[D:2 * D, :],
                    preferred_element_type=jnp.float32)
    h = jnp.maximum(h + bf1_ref[...], 0.0)
    s = jnp.dot(h, wf2_ref[...], preferred_element_type=jnp.float32)
    s = s + bf2_ref[...]
    o_ref[...] = 1.0 / (1.0 + jnp.exp(-s))


def _epilogue(v, w2, b2, wf1, bf1, wf2, bf2):
    return pl.pallas_call(
        _epilogue_body,
        out_shape=jax.ShapeDtypeStruct((1, 1), jnp.float32),
    )(v, w2, b2.reshape(1, D), wf1, bf1.reshape(1, D), wf2,
      bf2.reshape(1, 1))


def _pad_edges(ei):
    src = ei[0]
    dst = ei[1]
    pad = jnp.full((EP - E,), N, dtype=jnp.int32)
    return (jnp.concatenate([src, pad]), jnp.concatenate([dst, pad]))


def kernel(x1, edge_index1, x2, edge_index2,
           W0, b0, W1, b1, W2, b2, Wf1, bf1, Wf2, bf2):
    f32 = jnp.float32
    src1, dst1 = _pad_edges(edge_index1)
    src2, dst2 = _pad_edges(edge_index2)
    src = jnp.stack([src1, src2]).reshape(2, NS, EPS)
    dst = jnp.stack([dst1, dst2]).reshape(2, NS, EPS)
    src_off = jnp.stack([src1, src2 + NP]).reshape(2, NS, NCHS // 2, 2, CH)
    dst_ch = dst.reshape(2, NS, NCHS // 2, 2, CH)
    idxc = jnp.concatenate([src_off, dst_ch], axis=3)

    znp = jnp.zeros((NP,), f32)
    z640 = jnp.zeros((RPT, D), f32)
    _deg_kernel, _svec_kernel, _spmm_kernel = _sc_kernels()

    dinv = _psum(_deg_kernel(dst, znp), to_dinv=True)
    svec = _psum(_svec_kernel(src, dst, dinv, znp), to_dinv=False)
    node_mask = (jnp.arange(NP) < N)[None, :]
    c = jnp.where(node_mask, dinv * (svec + dinv), 0.0)

    X = jnp.zeros((2, NP, D), f32).at[:, :N, :].set(jnp.stack([x1, x2]))
    dcol = dinv[:, :, None]

    t0 = dcol * _matmul(X, W0)
    z0 = _spmm_kernel(t0.reshape(2 * NP, D), idxc, z640)
    h1 = jnp.maximum(dcol * (z0 + t0) + b0[None, None, :], 0.0)

    t1 = dcol * _matmul(h1, W1)
    z1 = _spmm_kernel(t1.reshape(2 * NP, D), idxc, z640)
    h2 = jnp.maximum(dcol * (z1 + t1) + b1[None, None, :], 0.0)

    v = _colsum(c[:, :, None] * h2)
    return _epilogue(v, W2, b2, Wf1, bf1, Wf2, bf2)
